# baseline (device time: 22123 ns/iter reference)
import jax
import jax.numpy as jnp
from jax import lax
from jax.experimental import pallas as pl
from jax.experimental.pallas import tpu as pltpu

N_LAYERS = 3
N_CH = 4


def kernel(x, Win0, Wout0, Win1, Wout1, Win2, Wout2):
    b, d_loc = x.shape
    _, h_loc = Win0.shape
    rb = b // N_CH

    def body(
        x_ref, win0_ref, wout0_ref, win1_ref, wout1_ref, win2_ref, wout2_ref,
        out_ref,
        h_send, x_send, h_recv, x_recv,
        h_send_sems, h_recv_sems, x_send_sems, x_recv_sems,
    ):
        my_x = lax.axis_index("x")
        my_y = lax.axis_index("y")
        y_peer = (my_x, 1 - my_y)
        x_peer = (1 - my_x, my_y)

        barrier_sem = pltpu.get_barrier_semaphore()
        for nbr in (y_peer, x_peer):
            pl.semaphore_signal(
                barrier_sem, inc=1,
                device_id=nbr, device_id_type=pl.DeviceIdType.MESH,
            )

        wins = [win0_ref, win1_ref, win2_ref]
        wouts = [wout0_ref, wout1_ref, wout2_ref]

        rdmas = []

        def exchange(send_buf, recv_buf, send_sems, recv_sems, i, c, peer):
            r = pltpu.make_async_remote_copy(
                src_ref=send_buf.at[i, c],
                dst_ref=recv_buf.at[i, c],
                send_sem=send_sems.at[i, c],
                recv_sem=recv_sems.at[i, c],
                device_id=peer,
                device_id_type=pl.DeviceIdType.MESH,
            )
            r.start()
            rdmas.append(r)
            return r

        x_parts = [x_ref[pl.ds(c * rb, rb), :] for c in range(N_CH)]
        ph = [None] * N_CH
        px = [None] * N_CH
        h_rd = [None] * N_CH
        x_rd = [None] * N_CH

        for c in range(N_CH):
            ph[c] = jnp.dot(
                x_parts[c], wins[0][...], preferred_element_type=jnp.float32
            )
            h_send[0, c] = ph[c].astype(jnp.bfloat16)
        pl.semaphore_wait(barrier_sem, 2)
        for c in range(N_CH):
            h_rd[c] = exchange(
                h_send, h_recv, h_send_sems, h_recv_sems, 0, c, y_peer
            )

        for i in range(N_LAYERS):
            for c in range(N_CH):
                h_rd[c].wait_recv()
                h_full = jnp.maximum(
                    ph[c] + h_recv[i, c].astype(jnp.float32), 0.0
                )
                px[c] = jnp.dot(
                    h_full, wouts[i][...],
                    preferred_element_type=jnp.float32,
                )
                x_send[i, c] = px[c].astype(jnp.bfloat16)
                x_rd[c] = exchange(
                    x_send, x_recv, x_send_sems, x_recv_sems, i, c, x_peer
                )
            for c in range(N_CH):
                x_rd[c].wait_recv()
                x_new = px[c] + x_recv[i, c].astype(jnp.float32)
                if i < N_LAYERS - 1:
                    x_parts[c] = x_new
                    ph[c] = jnp.dot(
                        x_new, wins[i + 1][...],
                        preferred_element_type=jnp.float32,
                    )
                    h_send[i + 1, c] = ph[c].astype(jnp.bfloat16)
                    h_rd[c] = exchange(
                        h_send, h_recv, h_send_sems, h_recv_sems,
                        i + 1, c, y_peer,
                    )
                else:
                    out_ref[pl.ds(c * rb, rb), :] = x_new
        for r in rdmas:
            r.wait_send()

    return pl.pallas_call(
        body,
        out_shape=jax.ShapeDtypeStruct((b, d_loc), jnp.float32),
        in_specs=[pl.BlockSpec(memory_space=pltpu.VMEM)] * 7,
        out_specs=pl.BlockSpec(memory_space=pltpu.VMEM),
        scratch_shapes=[
            pltpu.VMEM((N_LAYERS, N_CH, rb, h_loc), jnp.bfloat16),
            pltpu.VMEM((N_LAYERS, N_CH, rb, d_loc), jnp.bfloat16),
            pltpu.VMEM((N_LAYERS, N_CH, rb, h_loc), jnp.bfloat16),
            pltpu.VMEM((N_LAYERS, N_CH, rb, d_loc), jnp.bfloat16),
            pltpu.SemaphoreType.DMA((N_LAYERS, N_CH)),
            pltpu.SemaphoreType.DMA((N_LAYERS, N_CH)),
            pltpu.SemaphoreType.DMA((N_LAYERS, N_CH)),
            pltpu.SemaphoreType.DMA((N_LAYERS, N_CH)),
        ],
        compiler_params=pltpu.CompilerParams(collective_id=0),
    )(x, Win0, Wout0, Win1, Wout1, Win2, Wout2)
